# Initial kernel scaffold; baseline (speedup 1.0000x reference)
import jax
import jax.numpy as jnp
from jax.experimental import pallas as pl


def kernel(inputs, embeddings):
    raise NotImplementedError("write your pallas kernel here")


# speedup vs baseline: 1.0217x; 1.0217x over previous
import functools

import jax
import jax.numpy as jnp
from jax import lax
from jax.experimental import pallas as pl
from jax.experimental.pallas import tpu as pltpu
from jax.experimental.pallas import tpu_sc as plsc

EMBED_DIM = 32
NUM_WORKERS = 32
CHUNK = 128


@functools.lru_cache(maxsize=None)
def _make_gather(b_total):
    assert b_total % (NUM_WORKERS * CHUNK) == 0
    b_per_w = b_total // NUM_WORKERS
    nchunk = b_per_w // CHUNK

    mesh = plsc.VectorSubcoreMesh(core_axis_name="c", subcore_axis_name="s")

    @functools.partial(
        pl.kernel,
        mesh=mesh,
        out_type=jax.ShapeDtypeStruct((b_total, EMBED_DIM), jnp.float32),
        scratch_types=[
            pltpu.VMEM((nchunk, CHUNK), jnp.int32),
            pltpu.VMEM((CHUNK, EMBED_DIM), jnp.float32),
            pltpu.SemaphoreType.DMA,
        ],
        compiler_params=pltpu.CompilerParams(use_tc_tiling_on_sc=False),
    )
    def gather_kernel(table_hbm, idx_hbm, out_hbm, idx_v, rows_v, sem):
        wid = lax.axis_index("s") * 2 + lax.axis_index("c")
        base = wid * b_per_w
        pltpu.sync_copy(idx_hbm.at[pl.ds(wid * nchunk, nchunk)], idx_v)

        def body(j, carry):
            pltpu.async_copy(table_hbm.at[idx_v.at[j]], rows_v, sem).wait()
            pltpu.sync_copy(rows_v, out_hbm.at[pl.ds(base + j * CHUNK, CHUNK)])
            return carry

        lax.fori_loop(0, nchunk, body, 0)

    return gather_kernel


def kernel(inputs, embeddings):
    idx = inputs.astype(jnp.int32).reshape(-1, CHUNK)
    out = _make_gather(idx.size)(embeddings, idx)
    return out.reshape(inputs.shape + (embeddings.shape[-1],))


# speedup vs baseline: 1.1114x; 1.0878x over previous
import functools

import jax
import jax.numpy as jnp
from jax import lax
from jax.experimental import pallas as pl
from jax.experimental.pallas import tpu as pltpu
from jax.experimental.pallas import tpu_sc as plsc

EMBED_DIM = 32
NUM_WORKERS = 32
CHUNK = 128
NBUF = 10


@functools.lru_cache(maxsize=None)
def _make_gather(b_total):
    assert b_total % (NUM_WORKERS * CHUNK) == 0
    b_per_w = b_total // NUM_WORKERS
    nchunk = b_per_w // CHUNK
    ngroups = nchunk // NBUF
    npairs = ngroups // 2
    assert nchunk == ngroups * NBUF and ngroups == 2 * npairs

    mesh = plsc.VectorSubcoreMesh(core_axis_name="c", subcore_axis_name="s")

    @functools.partial(
        pl.kernel,
        mesh=mesh,
        out_type=jax.ShapeDtypeStruct((b_total, EMBED_DIM), jnp.float32),
        scratch_types=[
            pltpu.VMEM((nchunk, CHUNK), jnp.int32),
            pltpu.VMEM((NBUF, CHUNK, EMBED_DIM), jnp.float32),
            pltpu.VMEM((NBUF, CHUNK, EMBED_DIM), jnp.float32),
            pltpu.SemaphoreType.DMA,
            pltpu.SemaphoreType.DMA,
            pltpu.SemaphoreType.DMA,
            pltpu.SemaphoreType.DMA,
        ],
        compiler_params=pltpu.CompilerParams(use_tc_tiling_on_sc=False),
    )
    def gather_kernel(table_hbm, idx_hbm, out_hbm, idx_v, rows_a, rows_b,
                      gsem_a, gsem_b, ssem_a, ssem_b):
        wid = lax.axis_index("s") * 2 + lax.axis_index("c")
        base = wid * b_per_w
        pltpu.sync_copy(idx_hbm.at[pl.ds(wid * nchunk, nchunk)], idx_v)

        def gather_desc(chunk, rows, j, sem):
            return pltpu.make_async_copy(
                table_hbm.at[idx_v.at[chunk]], rows.at[j], sem)

        def store_desc(chunk, rows, j, sem):
            return pltpu.make_async_copy(
                rows.at[j], out_hbm.at[pl.ds(base + chunk * CHUNK, CHUNK)], sem)

        for j in range(NBUF):
            gather_desc(j, rows_a, j, gsem_a).start()

        def phase(g, rows_cur, gsem_cur, ssem_cur, rows_nxt, gsem_nxt,
                  issue_next):
            for j in range(NBUF):
                gather_desc(g * NBUF + j, rows_cur, j, gsem_cur).wait()
            if issue_next:
                for j in range(NBUF):
                    gather_desc((g + 1) * NBUF + j, rows_nxt, j,
                                gsem_nxt).start()
            for j in range(NBUF):
                store_desc(g * NBUF + j, rows_cur, j, ssem_cur).start()
            for j in range(NBUF):
                store_desc(g * NBUF + j, rows_cur, j, ssem_cur).wait()

        def body(p, carry):
            phase(2 * p, rows_a, gsem_a, ssem_a, rows_b, gsem_b, True)

            g = 2 * p + 1

            @pl.when(p < npairs - 1)
            def _():
                for j in range(NBUF):
                    gather_desc((g + 1) * NBUF + j, rows_a, j, gsem_a).start()

            for j in range(NBUF):
                gather_desc(g * NBUF + j, rows_b, j, gsem_b).wait()
            for j in range(NBUF):
                store_desc(g * NBUF + j, rows_b, j, ssem_b).start()
            for j in range(NBUF):
                store_desc(g * NBUF + j, rows_b, j, ssem_b).wait()
            return carry

        lax.fori_loop(0, npairs, body, 0)

    return gather_kernel


def kernel(inputs, embeddings):
    idx = inputs.astype(jnp.int32).reshape(-1, CHUNK)
    out = _make_gather(idx.size)(embeddings, idx)
    return out.reshape(inputs.shape + (embeddings.shape[-1],))
